# Initial kernel scaffold; baseline (speedup 1.0000x reference)
import jax
import jax.numpy as jnp
from jax.experimental import pallas as pl


def kernel(x, edge_index, Wk0, Wq0, Wv0, Ws0, b0, WkH, WqH, WvH, WsH, bH, Wout, bout):
    raise NotImplementedError("write your pallas kernel here")


# speedup vs baseline: 4.8521x; 4.8521x over previous
import jax
import jax.numpy as jnp
from jax import lax
from jax.experimental import pallas as pl
from jax.experimental.pallas import tpu as pltpu
from jax.experimental.pallas import tpu_sc as plsc

N = 10000
E = 320000
H = 128
OUT = 64
NL = 4

NC = 2
NS = 16
L = 16
NW = NC * NS

NP = 10240
CH = 120
EPW = E // NW
NCHUNK = 84
EPWP = NCHUNK * CH
RPT = NP // NS
RB = 80
NBLK = RPT // RB
VPR = H // L


BN = 1024


def _mm_body(x_ref, w_ref, b_ref, k_ref, q_ref, v_ref, s_ref):
    h = x_ref[...]
    kqvs = jnp.dot(h, w_ref[...], preferred_element_type=jnp.float32)
    kqvs = kqvs + b_ref[...]
    k_ref[...] = kqvs[:, 0 * H:1 * H]
    q_ref[...] = kqvs[:, 1 * H:2 * H]
    v_ref[...] = kqvs[:, 2 * H:3 * H]
    s_ref[...] = kqvs[:, 3 * H:4 * H]


def _combine_mm_body(a_ref, s_in_ref, w_ref, b_ref, k_ref, q_ref, v_ref, s_ref):
    h = a_ref[0] + a_ref[1] + s_in_ref[...]
    h = jnp.where(h > 0, h, 0.01 * h)
    kqvs = jnp.dot(h, w_ref[...], preferred_element_type=jnp.float32)
    kqvs = kqvs + b_ref[...]
    k_ref[...] = kqvs[:, 0 * H:1 * H]
    q_ref[...] = kqvs[:, 1 * H:2 * H]
    v_ref[...] = kqvs[:, 2 * H:3 * H]
    s_ref[...] = kqvs[:, 3 * H:4 * H]


def _final_body(a_ref, s_in_ref, w_ref, b_ref, o_ref):
    h = a_ref[0] + a_ref[1] + s_in_ref[...]
    h = jnp.where(h > 0, h, 0.01 * h)
    o_ref[...] = jnp.dot(h, w_ref[...], preferred_element_type=jnp.float32) + b_ref[...]


def _mm_first(x, wcat, bcat):
    grid = (NP // BN,)
    out = [jax.ShapeDtypeStruct((NP, H), jnp.float32)] * 4
    return pl.pallas_call(
        _mm_body,
        grid=grid,
        in_specs=[
            pl.BlockSpec((BN, H), lambda i: (i, 0)),
            pl.BlockSpec((H, 4 * H), lambda i: (0, 0)),
            pl.BlockSpec((1, 4 * H), lambda i: (0, 0)),
        ],
        out_specs=[pl.BlockSpec((BN, H), lambda i: (i, 0))] * 4,
        out_shape=out,
    )(x, wcat, bcat)


def _mm_hidden(agg, s_in, wcat, bcat):
    grid = (NP // BN,)
    out = [jax.ShapeDtypeStruct((NP, H), jnp.float32)] * 4
    return pl.pallas_call(
        _combine_mm_body,
        grid=grid,
        in_specs=[
            pl.BlockSpec((NC, BN, H), lambda i: (0, i, 0)),
            pl.BlockSpec((BN, H), lambda i: (i, 0)),
            pl.BlockSpec((H, 4 * H), lambda i: (0, 0)),
            pl.BlockSpec((1, 4 * H), lambda i: (0, 0)),
        ],
        out_specs=[pl.BlockSpec((BN, H), lambda i: (i, 0))] * 4,
        out_shape=out,
    )(agg, s_in, wcat, bcat)


def _mm_final(agg, s_in, wout, bout):
    grid = (NP // BN,)
    return pl.pallas_call(
        _final_body,
        grid=grid,
        in_specs=[
            pl.BlockSpec((NC, BN, H), lambda i: (0, i, 0)),
            pl.BlockSpec((BN, H), lambda i: (i, 0)),
            pl.BlockSpec((H, OUT), lambda i: (0, 0)),
            pl.BlockSpec((1, OUT), lambda i: (0, 0)),
        ],
        out_specs=pl.BlockSpec((BN, OUT), lambda i: (i, 0)),
        out_shape=jax.ShapeDtypeStruct((NP, OUT), jnp.float32),
    )(agg, s_in, wout, bout)


def _edge_body(src_hbm, dst_hbm, k_hbm, q_hbm, v_hbm, out_hbm,
               s_v, d_v, kd_v, qs_v, vs_v, agg_sh, sem):
    c = lax.axis_index("c")
    s = lax.axis_index("s")
    wid = s * NC + c

    def _zrow(r, _):
        for j in range(VPR):
            kd_v[r, pl.ds(j * L, L)] = jnp.zeros((L,), jnp.float32)
        return 0
    lax.fori_loop(0, RB, _zrow, 0)
    for blk in range(NBLK):
        pltpu.sync_copy(kd_v.at[pl.ds(0, RB)],
                        agg_sh.at[pl.ds(s * RPT + blk * RB, RB)])
    plsc.subcore_barrier()

    def _chunk(j, _):
        cp0 = pltpu.async_copy(src_hbm.at[wid, j], s_v, sem)
        cp0b = pltpu.async_copy(dst_hbm.at[wid, j], d_v, sem)
        cp0.wait()
        cp0b.wait()
        cp1 = pltpu.async_copy(k_hbm.at[d_v], kd_v, sem)
        cp2 = pltpu.async_copy(q_hbm.at[s_v], qs_v, sem)
        cp3 = pltpu.async_copy(v_hbm.at[s_v], vs_v, sem)
        cp1.wait()
        cp2.wait()
        cp3.wait()

        def _row(r, _):
            for jj in range(VPR):
                sl = pl.ds(jj * L, L)
                gate = kd_v[r, sl] + qs_v[r, sl]
                eta = 1.0 / (1.0 + jnp.exp(-gate))
                vs_v[r, sl] = eta * vs_v[r, sl]
            return 0
        lax.fori_loop(0, CH, _row, 0)
        pltpu.sync_copy(vs_v, agg_sh.at[d_v], add=True)
        return 0
    lax.fori_loop(0, NCHUNK, _chunk, 0)
    plsc.subcore_barrier()

    for blk in range(NBLK):
        row = s * RPT + blk * RB
        pltpu.sync_copy(agg_sh.at[pl.ds(row, RB)], kd_v.at[pl.ds(0, RB)])
        pltpu.sync_copy(kd_v.at[pl.ds(0, RB)], out_hbm.at[c, pl.ds(row, RB)])


_edge_pass = pl.kernel(
    _edge_body,
    out_type=jax.ShapeDtypeStruct((NC, NP, H), jnp.float32),
    mesh=plsc.VectorSubcoreMesh(core_axis_name="c", subcore_axis_name="s",
                                num_cores=NC, num_subcores=NS),
    scratch_types=[
        pltpu.VMEM((CH,), jnp.int32),
        pltpu.VMEM((CH,), jnp.int32),
        pltpu.VMEM((CH, H), jnp.float32),
        pltpu.VMEM((CH, H), jnp.float32),
        pltpu.VMEM((CH, H), jnp.float32),
        pltpu.VMEM_SHARED((NP, H), jnp.float32),
        pltpu.SemaphoreType.DMA,
    ],
)


def kernel(x, edge_index, Wk0, Wq0, Wv0, Ws0, b0, WkH, WqH, WvH, WsH, bH, Wout, bout):
    xp = jnp.pad(x, ((0, NP - N), (0, 0)))

    pad_idx = jnp.full((NW, EPWP - EPW), N, dtype=jnp.int32)
    src3 = jnp.concatenate([edge_index[0].reshape(NW, EPW), pad_idx],
                           axis=1).reshape(NW, NCHUNK, CH)
    dst3 = jnp.concatenate([edge_index[1].reshape(NW, EPW), pad_idx],
                           axis=1).reshape(NW, NCHUNK, CH)

    wcat0 = jnp.concatenate([Wk0, Wq0, Wv0, Ws0], axis=1)
    bcat0 = jnp.concatenate([jnp.zeros((3 * H,), jnp.float32), b0]).reshape(1, 4 * H)
    wcatH = [jnp.concatenate([WkH[l], WqH[l], WvH[l], WsH[l]], axis=1) for l in range(NL)]
    bcatH = [jnp.concatenate([jnp.zeros((3 * H,), jnp.float32), bH[l]]).reshape(1, 4 * H)
             for l in range(NL)]
    bout2 = bout.reshape(1, OUT)

    k, q, v, s = _mm_first(xp, wcat0, bcat0)
    agg = _edge_pass(src3, dst3, k, q, v)
    for _rep in range(3):
        for l in range(NL):
            k, q, v, s = _mm_hidden(agg, s, wcatH[l], bcatH[l])
            agg = _edge_pass(src3, dst3, k, q, v)
    return _mm_final(agg, s, Wout, bout2)[:N]
